# baseline (speedup 1.0000x reference)
import functools

import jax
import jax.numpy as jnp
from jax.experimental import pallas as pl
from jax.experimental.pallas import tpu as pltpu


def _prep_kernel(feat_ref, nfeat_ref, diff_ref, nf_ref):
    diff_ref[...] = (feat_ref[...] - nfeat_ref[...]).astype(jnp.bfloat16)
    x = nfeat_ref[...]
    for _ in range(2):
        ss = jnp.sum(x * x, axis=1, keepdims=True)
        ss_safe = jnp.where(ss > 0, ss, 1.0)
        x = jnp.where(x != 0, x / jnp.sqrt(ss_safe), 0.0)
    nf_ref[...] = x.astype(jnp.bfloat16)


def _fused_kernel(na_ref, edge_ref, diff_ref, nf_ref, pers_ref, abg_ref,
                  out_ref, g_scr, mask_scr, *, tm, half, inv_f):
    s = pl.program_id(0)

    @pl.when(s < half)
    def _phase1():
        i = s
        na = na_ref[...]
        g_scr[pl.ds(i * tm, tm), :] = jax.lax.dot_general(
            na.astype(jnp.bfloat16), diff_ref[...],
            (((1,), (0,)), ((), ())), preferred_element_type=jnp.float32,
        ).astype(jnp.bfloat16)
        mask_scr[pl.ds(i * tm, tm), :] = na.astype(jnp.int8)

    @pl.when(s >= half)
    def _phase2():
        i = s - half
        nf_i = nf_ref[pl.ds(i * tm, tm), :]
        g_i = g_scr[pl.ds(i * tm, tm), :]
        sim = jax.lax.dot_general(nf_i, nf_ref[...], (((1,), (1,)), ((), ())),
                                  preferred_element_type=jnp.float32)
        imp = jax.lax.dot_general(g_i, g_scr[...], (((1,), (1,)), ((), ())),
                                  preferred_element_type=jnp.float32)
        p = pers_ref[...]
        abg = abg_ref[...]
        pa = jnp.sum(p * abg[0:1, :], axis=1, keepdims=True)
        pb = jnp.sum(p * abg[1:2, :], axis=1, keepdims=True)
        pg = jnp.sum(p * abg[2:3, :], axis=1, keepdims=True)
        mask = mask_scr[pl.ds(i * tm, tm), :].astype(jnp.float32)
        out_ref[...] = (mask * sim * pa - edge_ref[...] * pb
                        + imp * (pg * inv_f))


def kernel(next_feature, next_action, feature, edge, alpha, beta, gamma,
           persona, time):
    n, f = feature.shape
    p = alpha.shape[0]
    persona_t = jax.lax.dynamic_index_in_dim(persona, time, axis=0,
                                             keepdims=False)
    abg = jnp.stack([alpha, beta, gamma])

    diff, nf = pl.pallas_call(
        _prep_kernel,
        out_shape=[
            jax.ShapeDtypeStruct((n, f), jnp.bfloat16),
            jax.ShapeDtypeStruct((n, f), jnp.bfloat16),
        ],
    )(feature, next_feature)

    tm = 256
    half = n // tm
    grid = (2 * half,)

    def _p1(s):
        return (jnp.minimum(s, half - 1), 0)

    def _p2(s):
        return (jnp.maximum(s - half, 0), 0)

    out = pl.pallas_call(
        functools.partial(_fused_kernel, tm=tm, half=half, inv_f=1.0 / f),
        grid=grid,
        in_specs=[
            pl.BlockSpec((tm, n), _p1),
            pl.BlockSpec((tm, n), _p2),
            pl.BlockSpec((n, f), lambda s: (0, 0)),
            pl.BlockSpec((n, f), lambda s: (0, 0)),
            pl.BlockSpec((tm, p), _p2),
            pl.BlockSpec((3, p), lambda s: (0, 0)),
        ],
        out_specs=pl.BlockSpec((tm, n), _p2),
        out_shape=jax.ShapeDtypeStruct((n, n), jnp.float32),
        scratch_shapes=[
            pltpu.VMEM((n, f), jnp.bfloat16),
            pltpu.VMEM((n, n), jnp.int8),
        ],
    )(next_action, edge, diff, nf, persona_t, abg)
    return out


# speedup vs baseline: 1.0871x; 1.0871x over previous
import functools

import jax
import jax.numpy as jnp
from jax.experimental import pallas as pl
from jax.experimental.pallas import tpu as pltpu


def _fused_kernel(na_ref, edge_ref, feat_ref, nfeat_ref, pers_ref, abg_ref,
                  out_ref, g_scr, nf_scr, diff_scr, mask_scr, *, tm, half,
                  inv_f):
    s = pl.program_id(0)

    @pl.when(s == 0)
    def _prep():
        diff_scr[...] = (feat_ref[...] - nfeat_ref[...]).astype(jnp.bfloat16)

    @pl.when(s < half)
    def _phase1():
        i = s
        na = na_ref[...]
        g_scr[pl.ds(i * tm, tm), :] = jax.lax.dot_general(
            na.astype(jnp.bfloat16), diff_scr[...],
            (((1,), (0,)), ((), ())), preferred_element_type=jnp.float32,
        ).astype(jnp.bfloat16)
        x = nfeat_ref[pl.ds(i * tm, tm), :]
        for _ in range(2):
            ss = jnp.sum(x * x, axis=1, keepdims=True)
            ss_safe = jnp.where(ss > 0, ss, 1.0)
            x = jnp.where(x != 0, x / jnp.sqrt(ss_safe), 0.0)
        nf_scr[pl.ds(i * tm, tm), :] = x.astype(jnp.bfloat16)
        mask_scr[pl.ds(i * tm, tm), :] = na.astype(jnp.int8)

    @pl.when(s >= half)
    def _phase2():
        i = s - half
        nf_i = nf_scr[pl.ds(i * tm, tm), :]
        g_i = g_scr[pl.ds(i * tm, tm), :]
        sim = jax.lax.dot_general(nf_i, nf_scr[...], (((1,), (1,)), ((), ())),
                                  preferred_element_type=jnp.float32)
        imp = jax.lax.dot_general(g_i, g_scr[...], (((1,), (1,)), ((), ())),
                                  preferred_element_type=jnp.float32)
        p = pers_ref[...]
        abg = abg_ref[...]
        pa = jnp.sum(p * abg[0:1, :], axis=1, keepdims=True)
        pb = jnp.sum(p * abg[1:2, :], axis=1, keepdims=True)
        pg = jnp.sum(p * abg[2:3, :], axis=1, keepdims=True)
        mask = mask_scr[pl.ds(i * tm, tm), :].astype(jnp.float32)
        out_ref[...] = (mask * sim * pa - edge_ref[...] * pb
                        + imp * (pg * inv_f))


def kernel(next_feature, next_action, feature, edge, alpha, beta, gamma,
           persona, time):
    n, f = feature.shape
    p = alpha.shape[0]
    persona_t = jax.lax.dynamic_index_in_dim(persona, time, axis=0,
                                             keepdims=False)
    abg = jnp.stack([alpha, beta, gamma])

    tm = 256
    half = n // tm
    grid = (2 * half,)

    def _p1(s):
        return (jnp.minimum(s, half - 1), 0)

    def _p2(s):
        return (jnp.maximum(s - half, 0), 0)

    out = pl.pallas_call(
        functools.partial(_fused_kernel, tm=tm, half=half, inv_f=1.0 / f),
        grid=grid,
        in_specs=[
            pl.BlockSpec((tm, n), _p1),
            pl.BlockSpec((tm, n), _p2),
            pl.BlockSpec((n, f), lambda s: (0, 0)),
            pl.BlockSpec((n, f), lambda s: (0, 0)),
            pl.BlockSpec((tm, p), _p2),
            pl.BlockSpec((3, p), lambda s: (0, 0)),
        ],
        out_specs=pl.BlockSpec((tm, n), _p2),
        out_shape=jax.ShapeDtypeStruct((n, n), jnp.float32),
        scratch_shapes=[
            pltpu.VMEM((n, f), jnp.bfloat16),
            pltpu.VMEM((n, f), jnp.bfloat16),
            pltpu.VMEM((n, f), jnp.bfloat16),
            pltpu.VMEM((n, n), jnp.int8),
        ],
        compiler_params=pltpu.CompilerParams(
            vmem_limit_bytes=100 * 1024 * 1024,
        ),
    )(next_action, edge, feature, next_feature, persona_t, abg)
    return out
